# baseline (speedup 1.0000x reference)
import jax
import jax.numpy as jnp
from jax.experimental import pallas as pl
from jax.experimental.pallas import tpu as pltpu

_T = 16
_H = 112
_W = 112
_C = 128
_D = 4
_WPAD = 144
_THRESH = 0.9


def _msd_body(x_ref, wi_ref, bi_ref, wb_ref, bb_ref, cw_ref, cb_ref,
              o_ref, a0, a1, a2, b0, b1, b2):
    nt = _H // _T
    trios = ((a0, a1, a2), (b0, b1, b2))

    for f0, f1, f2 in trios:
        f0[:, 0:17, :] = jnp.zeros((114, 17, _C), jnp.bfloat16)
        f1[:, 0:16, :] = jnp.zeros((114, 16, _C), jnp.bfloat16)
        f2[:, 0:15, :] = jnp.zeros((114, 15, _C), jnp.bfloat16)
        f2[:, 127:128, :] = jnp.zeros((114, 1, _C), jnp.bfloat16)
        for f in (f0, f1, f2):
            f[0, :, :] = jnp.zeros((_WPAD, _C), jnp.bfloat16)
            f[113, :, :] = jnp.zeros((_WPAD, _C), jnp.bfloat16)

    def store3(trio, r0, val):
        for kx in range(3):
            trio[kx][pl.ds(1 + r0, _T), pl.ds(17 - kx, _W), :] = val

    ncol = x_ref.shape[3]
    for rt in range(nt):
        r0 = rt * _T
        xs = x_ref[0, pl.ds(r0, _T), pl.ds(16, _W), :]
        y = jnp.dot(xs.reshape(_T * _W, ncol), wi_ref[:, :],
                    preferred_element_type=jnp.float32) + bi_ref[0]
        store3(trios[0], r0, y.astype(jnp.bfloat16).reshape(_T, _W, _C))

    out = jnp.zeros((_C,), jnp.float32)
    exited = jnp.zeros((), jnp.bool_)
    logits = None
    for d in range(_D):
        src = trios[d % 2]
        dst = trios[(d + 1) % 2]
        psum = jnp.zeros((_C,), jnp.float32)
        for rt in range(nt):
            r0 = rt * _T
            xcat = jnp.concatenate(
                [src[kx][pl.ds(r0 + ky, _T), pl.ds(16, _W), :]
                 .reshape(_T * _W, _C)
                 for ky in range(3) for kx in range(3)], axis=1)
            acc = jnp.dot(xcat, wb_ref[d],
                          preferred_element_type=jnp.float32)
            a = jnp.maximum(acc + bb_ref[d], 0.0)
            store3(dst, r0, a.astype(jnp.bfloat16).reshape(_T, _W, _C))
            psum = psum + jnp.sum(a, axis=0)
        pooled = psum * (1.0 / float(_H * _W))
        logits = (jnp.dot(pooled[None, :], cw_ref[d],
                          preferred_element_type=jnp.float32)[0]
                  + cb_ref[d])
        m = jnp.max(logits)
        conf = 1.0 / jnp.sum(jnp.exp(logits - m))
        take = jnp.logical_and(jnp.logical_not(exited), conf >= _THRESH)
        out = jnp.where(take, logits, out)
        exited = jnp.logical_or(exited, take)
    out = jnp.where(exited, out, logits)
    o_ref[0, 0, :] = out


def kernel(x, init_w, init_b, block_w, block_b, cls_w, cls_b):
    b = x.shape[0]
    cin = x.shape[1]
    nc = cls_w.shape[1]

    xt = jnp.transpose(x, (0, 2, 3, 1))
    xim = jnp.pad(xt, ((0, 0), (1, 1), (1, 1), (0, 0)))
    x_col = jnp.concatenate(
        [xim[:, ky:ky + _H, kx:kx + _W, :]
         for ky in range(3) for kx in range(3)], axis=3)
    x_col = jnp.pad(x_col, ((0, 0), (0, 0), (16, 16), (0, 0)))
    x_col = x_col.astype(jnp.bfloat16)

    wi = jnp.transpose(init_w[2], (2, 3, 1, 0)).reshape(9 * cin, _C)
    wi = wi.astype(jnp.bfloat16)
    bi = init_b[2].reshape(1, _C)
    wb = jnp.transpose(block_w[:, 2], (0, 3, 4, 2, 1)).reshape(_D, 9 * _C, _C)
    wb = wb.astype(jnp.bfloat16)
    bb = block_b[:, 2]
    cw = jnp.pad(jnp.transpose(cls_w, (0, 2, 1)),
                 ((0, 0), (0, 0), (0, _C - nc)))
    cb = jnp.pad(cls_b, ((0, 0), (0, _C - nc)), constant_values=-1e30)

    feat = lambda: pltpu.VMEM((114, _WPAD, _C), jnp.bfloat16)
    out_pad = pl.pallas_call(
        _msd_body,
        grid=(b,),
        in_specs=[
            pl.BlockSpec((1, _H, _WPAD, 9 * cin), lambda i: (i, 0, 0, 0)),
            pl.BlockSpec((9 * cin, _C), lambda i: (0, 0)),
            pl.BlockSpec((1, _C), lambda i: (0, 0)),
            pl.BlockSpec((_D, 9 * _C, _C), lambda i: (0, 0, 0)),
            pl.BlockSpec((_D, _C), lambda i: (0, 0)),
            pl.BlockSpec((_D, _C, _C), lambda i: (0, 0, 0)),
            pl.BlockSpec((_D, _C), lambda i: (0, 0)),
        ],
        out_specs=pl.BlockSpec((1, 1, _C), lambda i: (i, 0, 0)),
        out_shape=jax.ShapeDtypeStruct((b, 1, _C), jnp.float32),
        scratch_shapes=[feat() for _ in range(6)],
        compiler_params=pltpu.CompilerParams(
            dimension_semantics=("parallel",)),
    )(x_col, wi, bi, wb, bb, cw, cb)
    return out_pad[:, 0, :nc]


# speedup vs baseline: 1.3802x; 1.3802x over previous
import jax
import jax.numpy as jnp
from jax.experimental import pallas as pl
from jax.experimental.pallas import tpu as pltpu

_T = 16
_H = 112
_W = 112
_C = 128
_D = 4
_THRESH = 0.9


def _msd_body(x_ref, wi_ref, bi_ref, wb_ref, bb_ref, cw_ref, cb_ref,
              o_ref, fa, fb):
    nt = _H // _T

    for f in (fa, fb):
        f[0, :, :] = jnp.zeros((128, _C), jnp.float32)
        f[113, :, :] = jnp.zeros((128, _C), jnp.float32)
        f[:, 0:8, :] = jnp.zeros((114, 8, _C), jnp.float32)
        f[:, 120:128, :] = jnp.zeros((114, 8, _C), jnp.float32)

    ncol = x_ref.shape[3]
    for rt in range(nt):
        r0 = rt * _T
        xs = x_ref[0, pl.ds(r0, _T), pl.ds(8, _W), :]
        y = jnp.dot(xs.reshape(_T * _W, ncol), wi_ref[:, :],
                    preferred_element_type=jnp.float32) + bi_ref[0]
        fa[pl.ds(1 + r0, _T), pl.ds(8, _W), :] = y.reshape(_T, _W, _C)

    bufs = (fa, fb)
    out = jnp.zeros((_C,), jnp.float32)
    exited = jnp.zeros((), jnp.bool_)
    logits = None
    for d in range(_D):
        src = bufs[d % 2]
        dst = bufs[(d + 1) % 2]
        psum = jnp.zeros((_C,), jnp.float32)
        for rt in range(nt):
            r0 = rt * _T
            xcat = jnp.concatenate(
                [src[pl.ds(r0 + ky, _T), pl.ds(7 + kx, _W), :]
                 .reshape(_T * _W, _C)
                 for ky in range(3) for kx in range(3)], axis=1)
            acc = jnp.dot(xcat, wb_ref[d],
                          preferred_element_type=jnp.float32)
            a = jnp.maximum(acc + bb_ref[d], 0.0)
            dst[pl.ds(1 + r0, _T), pl.ds(8, _W), :] = a.reshape(_T, _W, _C)
            psum = psum + jnp.sum(a, axis=0)
        pooled = psum * (1.0 / float(_H * _W))
        logits = (jnp.dot(pooled[None, :], cw_ref[d],
                          preferred_element_type=jnp.float32)[0]
                  + cb_ref[d])
        m = jnp.max(logits)
        conf = 1.0 / jnp.sum(jnp.exp(logits - m))
        take = jnp.logical_and(jnp.logical_not(exited), conf >= _THRESH)
        out = jnp.where(take, logits, out)
        exited = jnp.logical_or(exited, take)
    out = jnp.where(exited, out, logits)
    o_ref[0, 0, :] = out


def kernel(x, init_w, init_b, block_w, block_b, cls_w, cls_b):
    b = x.shape[0]
    cin = x.shape[1]
    nc = cls_w.shape[1]

    xt = jnp.transpose(x, (0, 2, 3, 1))
    xim = jnp.pad(xt, ((0, 0), (1, 1), (1, 1), (0, 0)))
    x_col = jnp.concatenate(
        [xim[:, ky:ky + _H, kx:kx + _W, :]
         for ky in range(3) for kx in range(3)], axis=3)
    x_col = jnp.pad(x_col, ((0, 0), (0, 0), (8, 8), (0, 0)))

    wi = jnp.transpose(init_w[2], (2, 3, 1, 0)).reshape(9 * cin, _C)
    bi = init_b[2].reshape(1, _C)
    wb = jnp.transpose(block_w[:, 2], (0, 3, 4, 2, 1)).reshape(_D, 9 * _C, _C)
    bb = block_b[:, 2]
    cw = jnp.pad(jnp.transpose(cls_w, (0, 2, 1)),
                 ((0, 0), (0, 0), (0, _C - nc)))
    cb = jnp.pad(cls_b, ((0, 0), (0, _C - nc)), constant_values=-1e30)

    out_pad = pl.pallas_call(
        _msd_body,
        grid=(b,),
        in_specs=[
            pl.BlockSpec((1, _H, _C, 9 * cin), lambda i: (i, 0, 0, 0)),
            pl.BlockSpec((9 * cin, _C), lambda i: (0, 0)),
            pl.BlockSpec((1, _C), lambda i: (0, 0)),
            pl.BlockSpec((_D, 9 * _C, _C), lambda i: (0, 0, 0)),
            pl.BlockSpec((_D, _C), lambda i: (0, 0)),
            pl.BlockSpec((_D, _C, _C), lambda i: (0, 0, 0)),
            pl.BlockSpec((_D, _C), lambda i: (0, 0)),
        ],
        out_specs=pl.BlockSpec((1, 1, _C), lambda i: (i, 0, 0)),
        out_shape=jax.ShapeDtypeStruct((b, 1, _C), jnp.float32),
        scratch_shapes=[pltpu.VMEM((114, 128, _C), jnp.float32),
                        pltpu.VMEM((114, 128, _C), jnp.float32)],
        compiler_params=pltpu.CompilerParams(
            dimension_semantics=("parallel",)),
    )(x_col, wi, bi, wb, bb, cw, cb)
    return out_pad[:, 0, :nc]
